# baseline (speedup 1.0000x reference)
import functools
import math

import jax
import jax.numpy as jnp
from jax import lax
from jax.experimental import pallas as pl
from jax.experimental.pallas import tpu as pltpu
from jax.experimental.pallas import tpu_sc as plsc

N_NODES = 50000
N_EDGES = 800000
D = 64
W1 = 32
WC = 1
W2 = 16
NC, NS = 2, 16
NW = NC * NS
ROWS = 50048
RPT = ROWS // NS
CHUNK = 128
E_PAD = 819200
CPW = E_PAD // (NW * CHUNK)
LOG10 = math.log(10.0)
TCB = 2000
TCG = N_NODES // TCB

GRP = 5


@functools.lru_cache(maxsize=None)
def _make_seg(width, src_per_core):
    _mesh = plsc.VectorSubcoreMesh(core_axis_name="c", subcore_axis_name="s")
    cpw = E_PAD // ((NS if src_per_core else NW) * CHUNK)
    ng = cpw // GRP

    @functools.partial(
        pl.kernel,
        out_type=jax.ShapeDtypeStruct((NC, ROWS, width), jnp.float32),
        mesh=_mesh,
        scratch_types=(
            [pltpu.VMEM((GRP, CHUNK), jnp.int32)] * 4
            + [pltpu.VMEM((CHUNK, width), jnp.float32)] * GRP
            + [pltpu.VMEM_SHARED((ROWS, width), jnp.float32)]
            + [pltpu.SemaphoreType.DMA] * (4 + 2 * GRP)
        ),
        compiler_params=pltpu.CompilerParams(use_tc_tiling_on_sc=False),
    )
    def seg(tab, srcd, dstd, zer, out, *rest):
        sbufs, dbufs = rest[0:2], rest[2:4]
        rows = rest[4:4 + GRP]
        acc = rest[4 + GRP]
        sems = rest[5 + GRP:]
        ssem, dsem = sems[0:2], sems[2:4]
        gsem, csem = sems[4:4 + GRP], sems[4 + GRP:4 + 2 * GRP]
        c = lax.axis_index("c")
        s = lax.axis_index("s")
        w = s if src_per_core else s * NC + c
        base = w * cpw

        def src_slice(r0):
            return srcd.at[pl.ds(r0, GRP)]

        def stage(g, b):
            r0 = base + g * GRP
            pltpu.async_copy(src_slice(r0), sbufs[b], ssem[b])
            pltpu.async_copy(dstd.at[pl.ds(r0, GRP)], dbufs[b], dsem[b])

        stage(0, 0)
        pltpu.sync_copy(zer.at[pl.ds(s * RPT, RPT)], acc.at[pl.ds(s * RPT, RPT)])
        plsc.subcore_barrier()

        @pl.loop(0, ng, step=2)
        def group_pair(g0):
            for par in range(2):
                g = g0 + par
                sbuf, dbuf = sbufs[par], dbufs[par]
                pltpu.make_async_copy(src_slice(0), sbuf, ssem[par]).wait()
                pltpu.make_async_copy(dstd.at[pl.ds(0, GRP)], dbuf,
                                      dsem[par]).wait()
                if src_per_core:
                    for j in range(GRP):
                        for k in range(CHUNK // 16):
                            v = sbuf[j, pl.ds(k * 16, 16)]
                            sbuf[j, pl.ds(k * 16, 16)] = v + v + c
                for j in range(GRP):
                    @pl.when(g > 0)
                    def _():
                        pltpu.make_async_copy(rows[j], acc.at[dbuf.at[j]],
                                              csem[j]).wait()
                    pltpu.async_copy(tab.at[sbuf.at[j]], rows[j], gsem[j])

                @pl.when(g + 1 < ng)
                def _():
                    stage(g + 1, 1 - par)

                for j in range(GRP):
                    pltpu.make_async_copy(tab.at[sbuf.at[j]], rows[j],
                                          gsem[j]).wait()
                    pltpu.async_copy(rows[j], acc.at[dbuf.at[j]], csem[j],
                                     add=True)

        for j in range(GRP):
            pltpu.make_async_copy(rows[j], acc.at[dbufs[1].at[j]],
                                  csem[j]).wait()
        plsc.subcore_barrier()
        pltpu.sync_copy(acc.at[pl.ds(s * RPT, RPT)],
                        out.at[c, pl.ds(s * RPT, RPT)])

    return seg


@functools.lru_cache(maxsize=None)
def _make_cnt():
    _mesh = plsc.VectorSubcoreMesh(core_axis_name="c", subcore_axis_name="s")
    cpw = E_PAD // (NW * CHUNK)
    ng = cpw // GRP

    @functools.partial(
        pl.kernel,
        out_type=jax.ShapeDtypeStruct((NC, ROWS, WC), jnp.float32),
        mesh=_mesh,
        scratch_types=(
            [pltpu.VMEM((GRP, CHUNK), jnp.int32)] * 2
            + [pltpu.VMEM((CHUNK, WC), jnp.float32)]
            + [pltpu.VMEM_SHARED((ROWS, WC), jnp.float32)]
            + [pltpu.SemaphoreType.DMA] * (2 + GRP)
        ),
        compiler_params=pltpu.CompilerParams(use_tc_tiling_on_sc=False),
    )
    def cnt(dstd, ones, zer, out, *rest):
        dbufs = rest[0:2]
        ones_v = rest[2]
        acc = rest[3]
        dsem, csem = rest[4:6], rest[6:6 + GRP]
        c = lax.axis_index("c")
        s = lax.axis_index("s")
        w = s * NC + c
        base = w * cpw

        def stage(g, b):
            pltpu.async_copy(dstd.at[pl.ds(base + g * GRP, GRP)], dbufs[b],
                             dsem[b])

        stage(0, 0)
        pltpu.sync_copy(ones, ones_v)
        pltpu.sync_copy(zer.at[pl.ds(s * RPT, RPT)], acc.at[pl.ds(s * RPT, RPT)])
        plsc.subcore_barrier()

        @pl.loop(0, ng, step=2)
        def group_pair(g0):
            for par in range(2):
                g = g0 + par
                dbuf = dbufs[par]
                pltpu.make_async_copy(dstd.at[pl.ds(0, GRP)], dbuf,
                                      dsem[par]).wait()
                for j in range(GRP):
                    @pl.when(g > 0)
                    def _():
                        pltpu.make_async_copy(ones_v, acc.at[dbuf.at[j]],
                                              csem[j]).wait()
                    pltpu.async_copy(ones_v, acc.at[dbuf.at[j]], csem[j],
                                     add=True)

                @pl.when(g + 1 < ng)
                def _():
                    stage(g + 1, 1 - par)

        for j in range(GRP):
            pltpu.make_async_copy(ones_v, acc.at[dbufs[1].at[j]],
                                  csem[j]).wait()
        plsc.subcore_barrier()
        pltpu.sync_copy(acc.at[pl.ds(s * RPT, RPT)],
                        out.at[c, pl.ds(s * RPT, RPT)])

    return cnt


def _mid_body(s1a, s1b, c0, c1, x, wl1, bl1, wr1, g, b, wl2, wr2, bl2,
              p_out, r_out, ci_out):
    sums = jnp.concatenate([s1a[...], s1b[...]], axis=1)
    cnt = c0[...] + c1[...]
    ci = 1.0 / jnp.maximum(cnt, 1.0)
    h = (jnp.dot(sums * ci, wl1[...], preferred_element_type=jnp.float32)
         + bl1[...]
         + jnp.dot(x[...], wr1[...], preferred_element_type=jnp.float32))
    mu = jnp.mean(h, axis=1, keepdims=True)
    var = jnp.mean((h - mu) ** 2, axis=1, keepdims=True)
    h = (h - mu) / jnp.sqrt(var + 1e-5) * g[...] + b[...]
    h = jnp.maximum(h, 0.0)
    p_out[...] = jnp.dot(h, wl2[...], preferred_element_type=jnp.float32)
    r_out[...] = jnp.dot(h, wr2[...], preferred_element_type=jnp.float32) + bl2[...]
    ci_out[...] = jnp.broadcast_to(ci, (TCB, W2))


def _fin_body(s2a, s2b, r, ci, o):
    m = (s2a[...] + s2b[...]) * ci[...] + r[...]
    col = lax.broadcasted_iota(jnp.int32, m.shape, 1)
    o[...] = jnp.where(col < 4, jnp.tanh(m),
                       jnp.where(col == 4, jnp.exp(jnp.minimum(m, LOG10)), 0.0))


def kernel(x, edge_index, Wl1, bl1, Wr1, ln_g, ln_b, Wl2, bl2, Wr2):
    f32 = jnp.float32
    src = edge_index[0]
    dst = edge_index[1]

    tab1 = x.reshape(2 * N_NODES, W1)

    npad = E_PAD - N_EDGES
    src_p = jnp.concatenate(
        [src, (jnp.arange(npad, dtype=jnp.int32) * 37) % N_NODES]
    ).reshape(-1, CHUNK)
    dst_p = jnp.concatenate(
        [dst, jnp.full((npad,), N_NODES, jnp.int32)]).reshape(-1, CHUNK)

    cpart = _make_cnt()(dst_p, jnp.ones((CHUNK, WC), f32),
                        jnp.zeros((ROWS, WC), f32))
    s1 = _make_seg(W1, True)(tab1, src_p, dst_p, jnp.zeros((ROWS, W1), f32))

    wl2p = jnp.zeros((D, W2), f32).at[:, : Wl2.shape[1]].set(Wl2)
    wr2p = jnp.zeros((D, W2), f32).at[:, : Wr2.shape[1]].set(Wr2)
    bl2p = jnp.zeros((1, W2), f32).at[0, : bl2.shape[0]].set(bl2)

    full = lambda shape: pl.BlockSpec(shape, lambda i: (0,) * len(shape))
    p16, r16, ci16 = pl.pallas_call(
        _mid_body,
        grid=(TCG,),
        in_specs=[
            pl.BlockSpec((TCB, W1), lambda i: (i, 0)),
            pl.BlockSpec((TCB, W1), lambda i: (i, 0)),
            pl.BlockSpec((TCB, WC), lambda i: (i, 0)),
            pl.BlockSpec((TCB, WC), lambda i: (i, 0)),
            pl.BlockSpec((TCB, D), lambda i: (i, 0)),
            full((D, D)), full((1, D)), full((D, D)),
            full((1, D)), full((1, D)),
            full((D, W2)), full((D, W2)), full((1, W2)),
        ],
        out_specs=[
            pl.BlockSpec((TCB, W2), lambda i: (i, 0)),
            pl.BlockSpec((TCB, W2), lambda i: (i, 0)),
            pl.BlockSpec((TCB, W2), lambda i: (i, 0)),
        ],
        out_shape=[jax.ShapeDtypeStruct((N_NODES, W2), f32)] * 3,
    )(s1[0], s1[1], cpart[0], cpart[1], x, Wl1, bl1.reshape(1, D), Wr1,
      ln_g.reshape(1, D), ln_b.reshape(1, D), wl2p, wr2p, bl2p)

    s2 = _make_seg(W2, False)(p16, src_p, dst_p, jnp.zeros((ROWS, W2), f32))

    o = pl.pallas_call(
        _fin_body,
        grid=(TCG,),
        in_specs=[pl.BlockSpec((TCB, W2), lambda i: (i, 0))] * 4,
        out_specs=pl.BlockSpec((TCB, W2), lambda i: (i, 0)),
        out_shape=jax.ShapeDtypeStruct((N_NODES, W2), f32),
    )(s2[0], s2[1], r16, ci16)

    return (o[:, :4], o[:, 4])


# speedup vs baseline: 1.0062x; 1.0062x over previous
import functools
import math

import jax
import jax.numpy as jnp
from jax import lax
from jax.experimental import pallas as pl
from jax.experimental.pallas import tpu as pltpu
from jax.experimental.pallas import tpu_sc as plsc

N_NODES = 50000
N_EDGES = 800000
D = 64
W1 = 32
WC = 8
W2 = 16
NC, NS = 2, 16
NW = NC * NS
ROWS = 50048
RPT = ROWS // NS
CHUNK = 128
E_PAD = 819200
CPW = E_PAD // (NW * CHUNK)
LOG10 = math.log(10.0)
TCB = 2000
TCG = N_NODES // TCB

GRP = 5


@functools.lru_cache(maxsize=None)
def _make_seg(width, src_per_core):
    _mesh = plsc.VectorSubcoreMesh(core_axis_name="c", subcore_axis_name="s")
    cpw = E_PAD // ((NS if src_per_core else NW) * CHUNK)
    ng = cpw // GRP

    @functools.partial(
        pl.kernel,
        out_type=jax.ShapeDtypeStruct((NC, ROWS, width), jnp.float32),
        mesh=_mesh,
        scratch_types=(
            [pltpu.VMEM((GRP, CHUNK), jnp.int32)] * 4
            + [pltpu.VMEM((CHUNK, width), jnp.float32)] * GRP
            + [pltpu.VMEM_SHARED((ROWS, width), jnp.float32)]
            + [pltpu.SemaphoreType.DMA] * (4 + 2 * GRP)
        ),
        compiler_params=pltpu.CompilerParams(use_tc_tiling_on_sc=False),
    )
    def seg(tab, srcd, dstd, zer, out, *rest):
        sbufs, dbufs = rest[0:2], rest[2:4]
        rows = rest[4:4 + GRP]
        acc = rest[4 + GRP]
        sems = rest[5 + GRP:]
        ssem, dsem = sems[0:2], sems[2:4]
        gsem, csem = sems[4:4 + GRP], sems[4 + GRP:4 + 2 * GRP]
        c = lax.axis_index("c")
        s = lax.axis_index("s")
        w = s if src_per_core else s * NC + c
        base = w * cpw

        def src_slice(r0):
            return srcd.at[pl.ds(r0, GRP)]

        def stage(g, b):
            r0 = base + g * GRP
            pltpu.async_copy(src_slice(r0), sbufs[b], ssem[b])
            pltpu.async_copy(dstd.at[pl.ds(r0, GRP)], dbufs[b], dsem[b])

        stage(0, 0)
        pltpu.sync_copy(zer.at[pl.ds(s * RPT, RPT)], acc.at[pl.ds(s * RPT, RPT)])
        plsc.subcore_barrier()

        @pl.loop(0, ng, step=2)
        def group_pair(g0):
            for par in range(2):
                g = g0 + par
                sbuf, dbuf = sbufs[par], dbufs[par]
                pltpu.make_async_copy(src_slice(0), sbuf, ssem[par]).wait()
                pltpu.make_async_copy(dstd.at[pl.ds(0, GRP)], dbuf,
                                      dsem[par]).wait()
                if src_per_core:
                    for j in range(GRP):
                        for k in range(CHUNK // 16):
                            v = sbuf[j, pl.ds(k * 16, 16)]
                            sbuf[j, pl.ds(k * 16, 16)] = v + v + c
                for j in range(GRP):
                    @pl.when(g > 0)
                    def _():
                        pltpu.make_async_copy(rows[j], acc.at[dbuf.at[j]],
                                              csem[j]).wait()
                    pltpu.async_copy(tab.at[sbuf.at[j]], rows[j], gsem[j])

                @pl.when(g + 1 < ng)
                def _():
                    stage(g + 1, 1 - par)

                for j in range(GRP):
                    pltpu.make_async_copy(tab.at[sbuf.at[j]], rows[j],
                                          gsem[j]).wait()
                    pltpu.async_copy(rows[j], acc.at[dbuf.at[j]], csem[j],
                                     add=True)

        for j in range(GRP):
            pltpu.make_async_copy(rows[j], acc.at[dbufs[1].at[j]],
                                  csem[j]).wait()
        plsc.subcore_barrier()
        pltpu.sync_copy(acc.at[pl.ds(s * RPT, RPT)],
                        out.at[c, pl.ds(s * RPT, RPT)])

    return seg


@functools.lru_cache(maxsize=None)
def _make_cnt():
    _mesh = plsc.VectorSubcoreMesh(core_axis_name="c", subcore_axis_name="s")
    cpw = E_PAD // (NW * CHUNK)
    ng = cpw // GRP

    @functools.partial(
        pl.kernel,
        out_type=jax.ShapeDtypeStruct((NC, ROWS, WC), jnp.float32),
        mesh=_mesh,
        scratch_types=(
            [pltpu.VMEM((GRP, CHUNK), jnp.int32)] * 2
            + [pltpu.VMEM((CHUNK, WC), jnp.float32)]
            + [pltpu.VMEM_SHARED((ROWS, WC), jnp.float32)]
            + [pltpu.SemaphoreType.DMA] * (2 + GRP)
        ),
        compiler_params=pltpu.CompilerParams(use_tc_tiling_on_sc=False),
    )
    def cnt(dstd, ones, zer, out, *rest):
        dbufs = rest[0:2]
        ones_v = rest[2]
        acc = rest[3]
        dsem, csem = rest[4:6], rest[6:6 + GRP]
        c = lax.axis_index("c")
        s = lax.axis_index("s")
        w = s * NC + c
        base = w * cpw

        def stage(g, b):
            pltpu.async_copy(dstd.at[pl.ds(base + g * GRP, GRP)], dbufs[b],
                             dsem[b])

        stage(0, 0)
        pltpu.sync_copy(ones, ones_v)
        pltpu.sync_copy(zer.at[pl.ds(s * RPT, RPT)], acc.at[pl.ds(s * RPT, RPT)])
        plsc.subcore_barrier()

        @pl.loop(0, ng, step=2)
        def group_pair(g0):
            for par in range(2):
                g = g0 + par
                dbuf = dbufs[par]
                pltpu.make_async_copy(dstd.at[pl.ds(0, GRP)], dbuf,
                                      dsem[par]).wait()
                for j in range(GRP):
                    @pl.when(g > 0)
                    def _():
                        pltpu.make_async_copy(ones_v, acc.at[dbuf.at[j]],
                                              csem[j]).wait()
                    pltpu.async_copy(ones_v, acc.at[dbuf.at[j]], csem[j],
                                     add=True)

                @pl.when(g + 1 < ng)
                def _():
                    stage(g + 1, 1 - par)

        for j in range(GRP):
            pltpu.make_async_copy(ones_v, acc.at[dbufs[1].at[j]],
                                  csem[j]).wait()
        plsc.subcore_barrier()
        pltpu.sync_copy(acc.at[pl.ds(s * RPT, RPT)],
                        out.at[c, pl.ds(s * RPT, RPT)])

    return cnt


def _mid_body(s1a, s1b, c0, c1, x, wl1, bl1, wr1, g, b, wl2, wr2, bl2,
              p_out, r_out, ci_out):
    sums = jnp.concatenate([s1a[...], s1b[...]], axis=1)
    cnt = (c0[...] + c1[...])[:, :1]
    ci = 1.0 / jnp.maximum(cnt, 1.0)
    h = (jnp.dot(sums * ci, wl1[...], preferred_element_type=jnp.float32)
         + bl1[...]
         + jnp.dot(x[...], wr1[...], preferred_element_type=jnp.float32))
    mu = jnp.mean(h, axis=1, keepdims=True)
    var = jnp.mean((h - mu) ** 2, axis=1, keepdims=True)
    h = (h - mu) / jnp.sqrt(var + 1e-5) * g[...] + b[...]
    h = jnp.maximum(h, 0.0)
    p_out[...] = jnp.dot(h, wl2[...], preferred_element_type=jnp.float32)
    r_out[...] = jnp.dot(h, wr2[...], preferred_element_type=jnp.float32) + bl2[...]
    ci_out[...] = jnp.broadcast_to(ci, (TCB, W2))


def _fin_body(s2a, s2b, r, ci, o):
    m = (s2a[...] + s2b[...]) * ci[...] + r[...]
    col = lax.broadcasted_iota(jnp.int32, m.shape, 1)
    o[...] = jnp.where(col < 4, jnp.tanh(m),
                       jnp.where(col == 4, jnp.exp(jnp.minimum(m, LOG10)), 0.0))


def kernel(x, edge_index, Wl1, bl1, Wr1, ln_g, ln_b, Wl2, bl2, Wr2):
    f32 = jnp.float32
    src = edge_index[0]
    dst = edge_index[1]

    tab1 = x.reshape(2 * N_NODES, W1)

    npad = E_PAD - N_EDGES
    src_p = jnp.concatenate(
        [src, (jnp.arange(npad, dtype=jnp.int32) * 37) % N_NODES]
    ).reshape(-1, CHUNK)
    dst_p = jnp.concatenate(
        [dst, jnp.full((npad,), N_NODES, jnp.int32)]).reshape(-1, CHUNK)

    cpart = _make_cnt()(dst_p, jnp.ones((CHUNK, WC), f32),
                        jnp.zeros((ROWS, WC), f32))
    s1 = _make_seg(W1, True)(tab1, src_p, dst_p, jnp.zeros((ROWS, W1), f32))

    wl2p = jnp.zeros((D, W2), f32).at[:, : Wl2.shape[1]].set(Wl2)
    wr2p = jnp.zeros((D, W2), f32).at[:, : Wr2.shape[1]].set(Wr2)
    bl2p = jnp.zeros((1, W2), f32).at[0, : bl2.shape[0]].set(bl2)

    full = lambda shape: pl.BlockSpec(shape, lambda i: (0,) * len(shape))
    p16, r16, ci16 = pl.pallas_call(
        _mid_body,
        grid=(TCG,),
        in_specs=[
            pl.BlockSpec((TCB, W1), lambda i: (i, 0)),
            pl.BlockSpec((TCB, W1), lambda i: (i, 0)),
            pl.BlockSpec((TCB, WC), lambda i: (i, 0)),
            pl.BlockSpec((TCB, WC), lambda i: (i, 0)),
            pl.BlockSpec((TCB, D), lambda i: (i, 0)),
            full((D, D)), full((1, D)), full((D, D)),
            full((1, D)), full((1, D)),
            full((D, W2)), full((D, W2)), full((1, W2)),
        ],
        out_specs=[
            pl.BlockSpec((TCB, W2), lambda i: (i, 0)),
            pl.BlockSpec((TCB, W2), lambda i: (i, 0)),
            pl.BlockSpec((TCB, W2), lambda i: (i, 0)),
        ],
        out_shape=[jax.ShapeDtypeStruct((N_NODES, W2), f32)] * 3,
    )(s1[0], s1[1], cpart[0], cpart[1], x, Wl1, bl1.reshape(1, D), Wr1,
      ln_g.reshape(1, D), ln_b.reshape(1, D), wl2p, wr2p, bl2p)

    s2 = _make_seg(W2, False)(p16, src_p, dst_p, jnp.zeros((ROWS, W2), f32))

    o = pl.pallas_call(
        _fin_body,
        grid=(TCG,),
        in_specs=[pl.BlockSpec((TCB, W2), lambda i: (i, 0))] * 4,
        out_specs=pl.BlockSpec((TCB, W2), lambda i: (i, 0)),
        out_shape=jax.ShapeDtypeStruct((N_NODES, W2), f32),
    )(s2[0], s2[1], r16, ci16)

    return (o[:, :4], o[:, 4])


# speedup vs baseline: 1.0428x; 1.0364x over previous
import functools
import math

import jax
import jax.numpy as jnp
from jax import lax
from jax.experimental import pallas as pl
from jax.experimental.pallas import tpu as pltpu
from jax.experimental.pallas import tpu_sc as plsc

N_NODES = 50000
N_EDGES = 800000
D = 64
W1 = 32
WC = 8
W2 = 16
NC, NS = 2, 16
NW = NC * NS
ROWS = 50048
RPT = ROWS // NS
CHUNK = 128
E_PAD = 819200
CPW = E_PAD // (NW * CHUNK)
LOG10 = math.log(10.0)
TCB = 3128
TCG = ROWS // TCB

GRP = 5


@functools.lru_cache(maxsize=None)
def _make_seg(width, src_per_core, GRP=GRP):
    _mesh = plsc.VectorSubcoreMesh(core_axis_name="c", subcore_axis_name="s")
    cpw = E_PAD // ((NS if src_per_core else NW) * CHUNK)
    ng = cpw // GRP

    @functools.partial(
        pl.kernel,
        out_type=jax.ShapeDtypeStruct((NC, ROWS, width), jnp.float32),
        mesh=_mesh,
        scratch_types=(
            [pltpu.VMEM((GRP, CHUNK), jnp.int32)] * 4
            + [pltpu.VMEM((CHUNK, width), jnp.float32)] * GRP
            + [pltpu.VMEM_SHARED((ROWS, width), jnp.float32)]
            + [pltpu.SemaphoreType.DMA] * (4 + 2 * GRP)
        ),
        compiler_params=pltpu.CompilerParams(use_tc_tiling_on_sc=False),
    )
    def seg(tab, srcd, dstd, zer, out, *rest):
        sbufs, dbufs = rest[0:2], rest[2:4]
        rows = rest[4:4 + GRP]
        acc = rest[4 + GRP]
        sems = rest[5 + GRP:]
        ssem, dsem = sems[0:2], sems[2:4]
        gsem, csem = sems[4:4 + GRP], sems[4 + GRP:4 + 2 * GRP]
        c = lax.axis_index("c")
        s = lax.axis_index("s")
        w = s if src_per_core else s * NC + c
        base = w * cpw

        def src_slice(r0):
            return srcd.at[pl.ds(r0, GRP)]

        def stage(g, b):
            r0 = base + g * GRP
            pltpu.async_copy(src_slice(r0), sbufs[b], ssem[b])
            pltpu.async_copy(dstd.at[pl.ds(r0, GRP)], dbufs[b], dsem[b])

        stage(0, 0)
        pltpu.sync_copy(zer.at[pl.ds(s * RPT, RPT)], acc.at[pl.ds(s * RPT, RPT)])
        plsc.subcore_barrier()

        @pl.loop(0, ng, step=2)
        def group_pair(g0):
            for par in range(2):
                g = g0 + par
                sbuf, dbuf = sbufs[par], dbufs[par]
                pltpu.make_async_copy(src_slice(0), sbuf, ssem[par]).wait()
                pltpu.make_async_copy(dstd.at[pl.ds(0, GRP)], dbuf,
                                      dsem[par]).wait()
                if src_per_core:
                    for j in range(GRP):
                        for k in range(CHUNK // 16):
                            v = sbuf[j, pl.ds(k * 16, 16)]
                            sbuf[j, pl.ds(k * 16, 16)] = v + v + c
                for j in range(GRP):
                    @pl.when(g > 0)
                    def _():
                        pltpu.make_async_copy(rows[j], acc.at[dbuf.at[j]],
                                              csem[j]).wait()
                    pltpu.async_copy(tab.at[sbuf.at[j]], rows[j], gsem[j])

                @pl.when(g + 1 < ng)
                def _():
                    stage(g + 1, 1 - par)

                for j in range(GRP):
                    pltpu.make_async_copy(tab.at[sbuf.at[j]], rows[j],
                                          gsem[j]).wait()
                    pltpu.async_copy(rows[j], acc.at[dbuf.at[j]], csem[j],
                                     add=True)

        for j in range(GRP):
            pltpu.make_async_copy(rows[j], acc.at[dbufs[1].at[j]],
                                  csem[j]).wait()
        plsc.subcore_barrier()
        pltpu.sync_copy(acc.at[pl.ds(s * RPT, RPT)],
                        out.at[c, pl.ds(s * RPT, RPT)])

    return seg


@functools.lru_cache(maxsize=None)
def _make_cnt():
    _mesh = plsc.VectorSubcoreMesh(core_axis_name="c", subcore_axis_name="s")
    cpw = E_PAD // (NW * CHUNK)
    ng = cpw // GRP

    @functools.partial(
        pl.kernel,
        out_type=jax.ShapeDtypeStruct((NC, ROWS, WC), jnp.float32),
        mesh=_mesh,
        scratch_types=(
            [pltpu.VMEM((GRP, CHUNK), jnp.int32)] * 2
            + [pltpu.VMEM((CHUNK, WC), jnp.float32)]
            + [pltpu.VMEM_SHARED((ROWS, WC), jnp.float32)]
            + [pltpu.SemaphoreType.DMA] * (2 + GRP)
        ),
        compiler_params=pltpu.CompilerParams(use_tc_tiling_on_sc=False),
    )
    def cnt(dstd, ones, zer, out, *rest):
        dbufs = rest[0:2]
        ones_v = rest[2]
        acc = rest[3]
        dsem, csem = rest[4:6], rest[6:6 + GRP]
        c = lax.axis_index("c")
        s = lax.axis_index("s")
        w = s * NC + c
        base = w * cpw

        def stage(g, b):
            pltpu.async_copy(dstd.at[pl.ds(base + g * GRP, GRP)], dbufs[b],
                             dsem[b])

        stage(0, 0)
        pltpu.sync_copy(ones, ones_v)
        pltpu.sync_copy(zer.at[pl.ds(s * RPT, RPT)], acc.at[pl.ds(s * RPT, RPT)])
        plsc.subcore_barrier()

        @pl.loop(0, ng, step=2)
        def group_pair(g0):
            for par in range(2):
                g = g0 + par
                dbuf = dbufs[par]
                pltpu.make_async_copy(dstd.at[pl.ds(0, GRP)], dbuf,
                                      dsem[par]).wait()
                for j in range(GRP):
                    @pl.when(g > 0)
                    def _():
                        pltpu.make_async_copy(ones_v, acc.at[dbuf.at[j]],
                                              csem[j]).wait()
                    pltpu.async_copy(ones_v, acc.at[dbuf.at[j]], csem[j],
                                     add=True)

                @pl.when(g + 1 < ng)
                def _():
                    stage(g + 1, 1 - par)

        for j in range(GRP):
            pltpu.make_async_copy(ones_v, acc.at[dbufs[1].at[j]],
                                  csem[j]).wait()
        plsc.subcore_barrier()
        pltpu.sync_copy(acc.at[pl.ds(s * RPT, RPT)],
                        out.at[c, pl.ds(s * RPT, RPT)])

    return cnt


def _mid_body(s1a, s1b, c0, c1, x, wl1, bl1, wr1, g, b, wl2, wr2, bl2,
              p_out, r_out, ci_out):
    sums = jnp.concatenate([s1a[...], s1b[...]], axis=1)
    cnt = (c0[...] + c1[...])[:, :1]
    ci = 1.0 / jnp.maximum(cnt, 1.0)
    h = (jnp.dot(sums * ci, wl1[...], preferred_element_type=jnp.float32)
         + bl1[...]
         + jnp.dot(x[...], wr1[...], preferred_element_type=jnp.float32))
    mu = jnp.mean(h, axis=1, keepdims=True)
    var = jnp.mean((h - mu) ** 2, axis=1, keepdims=True)
    h = (h - mu) / jnp.sqrt(var + 1e-5) * g[...] + b[...]
    h = jnp.maximum(h, 0.0)
    p_out[...] = jnp.dot(h, wl2[...], preferred_element_type=jnp.float32)
    r_out[...] = jnp.dot(h, wr2[...], preferred_element_type=jnp.float32) + bl2[...]
    ci_out[...] = jnp.broadcast_to(ci, (TCB, W2))


def _fin_body(s2a, s2b, r, ci, o):
    m = (s2a[...] + s2b[...]) * ci[...] + r[...]
    col = lax.broadcasted_iota(jnp.int32, m.shape, 1)
    o[...] = jnp.where(col < 4, jnp.tanh(m),
                       jnp.where(col == 4, jnp.exp(jnp.minimum(m, LOG10)), 0.0))


def kernel(x, edge_index, Wl1, bl1, Wr1, ln_g, ln_b, Wl2, bl2, Wr2):
    f32 = jnp.float32
    src = edge_index[0]
    dst = edge_index[1]

    xpad = jnp.zeros((ROWS, D), f32).at[:N_NODES].set(x)
    tab1 = xpad.reshape(2 * ROWS, W1)

    npad = E_PAD - N_EDGES
    src_p = jnp.concatenate(
        [src, (jnp.arange(npad, dtype=jnp.int32) * 37) % N_NODES]
    ).reshape(-1, CHUNK)
    dst_p = jnp.concatenate(
        [dst, jnp.full((npad,), N_NODES, jnp.int32)]).reshape(-1, CHUNK)

    cpart = _make_cnt()(dst_p, jnp.ones((CHUNK, WC), f32),
                        jnp.zeros((ROWS, WC), f32))
    s1 = _make_seg(W1, True)(tab1, src_p, dst_p, jnp.zeros((ROWS, W1), f32))

    wl2p = jnp.zeros((D, W2), f32).at[:, : Wl2.shape[1]].set(Wl2)
    wr2p = jnp.zeros((D, W2), f32).at[:, : Wr2.shape[1]].set(Wr2)
    bl2p = jnp.zeros((1, W2), f32).at[0, : bl2.shape[0]].set(bl2)

    full = lambda shape: pl.BlockSpec(shape, lambda i: (0,) * len(shape))
    p16, r16, ci16 = pl.pallas_call(
        _mid_body,
        grid=(TCG,),
        in_specs=[
            pl.BlockSpec((TCB, W1), lambda i: (i, 0)),
            pl.BlockSpec((TCB, W1), lambda i: (i, 0)),
            pl.BlockSpec((TCB, WC), lambda i: (i, 0)),
            pl.BlockSpec((TCB, WC), lambda i: (i, 0)),
            pl.BlockSpec((TCB, D), lambda i: (i, 0)),
            full((D, D)), full((1, D)), full((D, D)),
            full((1, D)), full((1, D)),
            full((D, W2)), full((D, W2)), full((1, W2)),
        ],
        out_specs=[
            pl.BlockSpec((TCB, W2), lambda i: (i, 0)),
            pl.BlockSpec((TCB, W2), lambda i: (i, 0)),
            pl.BlockSpec((TCB, W2), lambda i: (i, 0)),
        ],
        out_shape=[jax.ShapeDtypeStruct((ROWS, W2), f32)] * 3,
    )(s1[0], s1[1], cpart[0], cpart[1], xpad, Wl1, bl1.reshape(1, D), Wr1,
      ln_g.reshape(1, D), ln_b.reshape(1, D), wl2p, wr2p, bl2p)

    s2 = _make_seg(W2, False)(p16, src_p, dst_p, jnp.zeros((ROWS, W2), f32))

    o = pl.pallas_call(
        _fin_body,
        grid=(TCG,),
        in_specs=[pl.BlockSpec((TCB, W2), lambda i: (i, 0))] * 4,
        out_specs=pl.BlockSpec((TCB, W2), lambda i: (i, 0)),
        out_shape=jax.ShapeDtypeStruct((ROWS, W2), f32),
    )(s2[0], s2[1], r16, ci16)

    y = o[:N_NODES]
    return (y[:, :4], y[:, 4])


# speedup vs baseline: 1.0689x; 1.0251x over previous
import functools
import math

import jax
import jax.numpy as jnp
from jax import lax
from jax.experimental import pallas as pl
from jax.experimental.pallas import tpu as pltpu
from jax.experimental.pallas import tpu_sc as plsc

N_NODES = 50000
N_EDGES = 800000
D = 64
W1 = 32
WC = 8
W2 = 16
NC, NS = 2, 16
NW = NC * NS
ROWS = 50176
RPT = ROWS // NS
CHUNK = 128
E_PAD = 819200
CPW = E_PAD // (NW * CHUNK)
LOG10 = math.log(10.0)
TCB = 3136
TCG = ROWS // TCB

GRP = 5


@functools.lru_cache(maxsize=None)
def _make_seg(width, src_per_core, GRP=GRP):
    _mesh = plsc.VectorSubcoreMesh(core_axis_name="c", subcore_axis_name="s")
    cpw = E_PAD // ((NS if src_per_core else NW) * CHUNK)
    ng = cpw // GRP

    @functools.partial(
        pl.kernel,
        out_type=jax.ShapeDtypeStruct((NC, ROWS, width), jnp.float32),
        mesh=_mesh,
        scratch_types=(
            [pltpu.VMEM((GRP, CHUNK), jnp.int32)] * 4
            + [pltpu.VMEM((CHUNK, width), jnp.float32)] * GRP
            + [pltpu.VMEM_SHARED((ROWS, width), jnp.float32)]
            + [pltpu.SemaphoreType.DMA] * (4 + 2 * GRP)
        ),
        compiler_params=pltpu.CompilerParams(use_tc_tiling_on_sc=False),
    )
    def seg(tab, srcd, dstd, zer, out, *rest):
        sbufs, dbufs = rest[0:2], rest[2:4]
        rows = rest[4:4 + GRP]
        acc = rest[4 + GRP]
        sems = rest[5 + GRP:]
        ssem, dsem = sems[0:2], sems[2:4]
        gsem, csem = sems[4:4 + GRP], sems[4 + GRP:4 + 2 * GRP]
        c = lax.axis_index("c")
        s = lax.axis_index("s")
        w = s if src_per_core else s * NC + c
        base = w * cpw

        def src_slice(r0):
            return srcd.at[pl.ds(r0, GRP)]

        def stage(g, b):
            r0 = base + g * GRP
            pltpu.async_copy(src_slice(r0), sbufs[b], ssem[b])
            pltpu.async_copy(dstd.at[pl.ds(r0, GRP)], dbufs[b], dsem[b])

        stage(0, 0)
        pltpu.sync_copy(zer.at[pl.ds(s * RPT, RPT)], acc.at[pl.ds(s * RPT, RPT)])
        plsc.subcore_barrier()

        @pl.loop(0, ng, step=2)
        def group_pair(g0):
            for par in range(2):
                g = g0 + par
                sbuf, dbuf = sbufs[par], dbufs[par]
                pltpu.make_async_copy(src_slice(0), sbuf, ssem[par]).wait()
                pltpu.make_async_copy(dstd.at[pl.ds(0, GRP)], dbuf,
                                      dsem[par]).wait()
                if src_per_core:
                    for j in range(GRP):
                        for k in range(CHUNK // 16):
                            v = sbuf[j, pl.ds(k * 16, 16)]
                            sbuf[j, pl.ds(k * 16, 16)] = v + v + c
                for j in range(GRP):
                    @pl.when(g > 0)
                    def _():
                        pltpu.make_async_copy(rows[j], acc.at[dbuf.at[j]],
                                              csem[j]).wait()
                    pltpu.async_copy(tab.at[sbuf.at[j]], rows[j], gsem[j])

                @pl.when(g + 1 < ng)
                def _():
                    stage(g + 1, 1 - par)

                for j in range(GRP):
                    pltpu.make_async_copy(tab.at[sbuf.at[j]], rows[j],
                                          gsem[j]).wait()
                    pltpu.async_copy(rows[j], acc.at[dbuf.at[j]], csem[j],
                                     add=True)

        for j in range(GRP):
            pltpu.make_async_copy(rows[j], acc.at[dbufs[1].at[j]],
                                  csem[j]).wait()
        plsc.subcore_barrier()
        pltpu.sync_copy(acc.at[pl.ds(s * RPT, RPT)],
                        out.at[c, pl.ds(s * RPT, RPT)])

    return seg


@functools.lru_cache(maxsize=None)
def _make_cnt():
    _mesh = plsc.VectorSubcoreMesh(core_axis_name="c", subcore_axis_name="s")
    cpw = E_PAD // (NW * CHUNK)
    ng = cpw // GRP

    @functools.partial(
        pl.kernel,
        out_type=jax.ShapeDtypeStruct((NC, ROWS, WC), jnp.float32),
        mesh=_mesh,
        scratch_types=(
            [pltpu.VMEM((GRP, CHUNK), jnp.int32)] * 2
            + [pltpu.VMEM((CHUNK, WC), jnp.float32)]
            + [pltpu.VMEM_SHARED((ROWS, WC), jnp.float32)]
            + [pltpu.SemaphoreType.DMA] * (2 + GRP)
        ),
        compiler_params=pltpu.CompilerParams(use_tc_tiling_on_sc=False),
    )
    def cnt(dstd, ones, zer, out, *rest):
        dbufs = rest[0:2]
        ones_v = rest[2]
        acc = rest[3]
        dsem, csem = rest[4:6], rest[6:6 + GRP]
        c = lax.axis_index("c")
        s = lax.axis_index("s")
        w = s * NC + c
        base = w * cpw

        def stage(g, b):
            pltpu.async_copy(dstd.at[pl.ds(base + g * GRP, GRP)], dbufs[b],
                             dsem[b])

        stage(0, 0)
        pltpu.sync_copy(ones, ones_v)
        pltpu.sync_copy(zer.at[pl.ds(s * RPT, RPT)], acc.at[pl.ds(s * RPT, RPT)])
        plsc.subcore_barrier()

        @pl.loop(0, ng, step=2)
        def group_pair(g0):
            for par in range(2):
                g = g0 + par
                dbuf = dbufs[par]
                pltpu.make_async_copy(dstd.at[pl.ds(0, GRP)], dbuf,
                                      dsem[par]).wait()
                for j in range(GRP):
                    @pl.when(g > 0)
                    def _():
                        pltpu.make_async_copy(ones_v, acc.at[dbuf.at[j]],
                                              csem[j]).wait()
                    pltpu.async_copy(ones_v, acc.at[dbuf.at[j]], csem[j],
                                     add=True)

                @pl.when(g + 1 < ng)
                def _():
                    stage(g + 1, 1 - par)

        for j in range(GRP):
            pltpu.make_async_copy(ones_v, acc.at[dbufs[1].at[j]],
                                  csem[j]).wait()
        plsc.subcore_barrier()
        pltpu.sync_copy(acc.at[pl.ds(s * RPT, RPT)],
                        out.at[c, pl.ds(s * RPT, RPT)])

    return cnt


def _mid_body(s1a, s1b, c0, c1, x, wl1, bl1, wr1, g, b, wl2, wr2, bl2,
              q_out):
    sums = jnp.concatenate([s1a[...], s1b[...]], axis=1)
    cnt = (c0[...] + c1[...])[:, :1]
    ci = 1.0 / jnp.maximum(cnt, 1.0)
    h = (jnp.dot(sums * ci, wl1[...], preferred_element_type=jnp.float32)
         + bl1[...]
         + jnp.dot(x[...], wr1[...], preferred_element_type=jnp.float32))
    mu = jnp.mean(h, axis=1, keepdims=True)
    var = jnp.mean((h - mu) ** 2, axis=1, keepdims=True)
    h = (h - mu) / jnp.sqrt(var + 1e-5) * g[...] + b[...]
    h = jnp.maximum(h, 0.0)
    p = jnp.dot(h, wl2[...], preferred_element_type=jnp.float32)
    r = jnp.dot(h, wr2[...], preferred_element_type=jnp.float32) + bl2[...]
    q_out[...] = jnp.concatenate(
        [p, r, jnp.broadcast_to(ci, (TCB, W2)),
         jnp.zeros((TCB, 128 - 3 * W2), jnp.float32)], axis=1)


def _fin_body(s2a, s2b, q, o):
    qv = q[...]
    m = (s2a[...] + s2b[...]) * qv[:, 2 * W2:3 * W2] + qv[:, W2:2 * W2]
    col = lax.broadcasted_iota(jnp.int32, m.shape, 1)
    o[...] = jnp.where(col < 4, jnp.tanh(m),
                       jnp.where(col == 4, jnp.exp(jnp.minimum(m, LOG10)), 0.0))


def kernel(x, edge_index, Wl1, bl1, Wr1, ln_g, ln_b, Wl2, bl2, Wr2):
    f32 = jnp.float32
    src = edge_index[0]
    dst = edge_index[1]

    xpad = jnp.zeros((ROWS, D), f32).at[:N_NODES].set(x)
    tab1 = xpad.reshape(2 * ROWS, W1)

    npad = E_PAD - N_EDGES
    src_p = jnp.concatenate(
        [src, (jnp.arange(npad, dtype=jnp.int32) * 37) % N_NODES]
    ).reshape(-1, CHUNK)
    dst_p = jnp.concatenate(
        [dst, jnp.full((npad,), N_NODES, jnp.int32)]).reshape(-1, CHUNK)

    cpart = _make_cnt()(dst_p, jnp.ones((CHUNK, WC), f32),
                        jnp.zeros((ROWS, WC), f32))
    s1 = _make_seg(W1, True)(tab1, src_p, dst_p, jnp.zeros((ROWS, W1), f32))

    wl2p = jnp.zeros((D, W2), f32).at[:, : Wl2.shape[1]].set(Wl2)
    wr2p = jnp.zeros((D, W2), f32).at[:, : Wr2.shape[1]].set(Wr2)
    bl2p = jnp.zeros((1, W2), f32).at[0, : bl2.shape[0]].set(bl2)

    full = lambda shape: pl.BlockSpec(shape, lambda i: (0,) * len(shape))
    q = pl.pallas_call(
        _mid_body,
        grid=(TCG,),
        in_specs=[
            pl.BlockSpec((TCB, W1), lambda i: (i, 0)),
            pl.BlockSpec((TCB, W1), lambda i: (i, 0)),
            pl.BlockSpec((TCB, WC), lambda i: (i, 0)),
            pl.BlockSpec((TCB, WC), lambda i: (i, 0)),
            pl.BlockSpec((TCB, D), lambda i: (i, 0)),
            full((D, D)), full((1, D)), full((D, D)),
            full((1, D)), full((1, D)),
            full((D, W2)), full((D, W2)), full((1, W2)),
        ],
        out_specs=pl.BlockSpec((TCB, 128), lambda i: (i, 0)),
        out_shape=jax.ShapeDtypeStruct((ROWS, 128), f32),
    )(s1[0], s1[1], cpart[0], cpart[1], xpad, Wl1, bl1.reshape(1, D),
      Wr1, ln_g.reshape(1, D), ln_b.reshape(1, D), wl2p, wr2p, bl2p)

    s2 = _make_seg(W2, False)(q[:, :W2], src_p, dst_p,
                              jnp.zeros((ROWS, W2), f32))

    o = pl.pallas_call(
        _fin_body,
        grid=(TCG,),
        in_specs=[
            pl.BlockSpec((TCB, W2), lambda i: (i, 0)),
            pl.BlockSpec((TCB, W2), lambda i: (i, 0)),
            pl.BlockSpec((TCB, 128), lambda i: (i, 0)),
        ],
        out_specs=pl.BlockSpec((TCB, W2), lambda i: (i, 0)),
        out_shape=jax.ShapeDtypeStruct((ROWS, W2), f32),
    )(s2[0], s2[1], q)

    y = o[:N_NODES]
    return (y[:, :4], y[:, 4])
